# baseline (speedup 1.0000x reference)
import functools

import jax
import jax.numpy as jnp
from jax import lax
from jax.experimental import pallas as pl
from jax.experimental.pallas import tpu as pltpu
from jax.experimental.pallas import tpu_sc as plsc

N, K, D = 1024, 32, 128
BLK = 128
NT = 768
NS = N - NT
NW = 32
NPW = NS // NW
CH = 4
NCH = NPW // CH
PL_ = 512


def _prep_body(W_ref, Wa_ref, ssrc_ref, stgt_ref, ba_ref, out_ref):
    W = W_ref[...]
    Wa = Wa_ref[...]
    svec = ssrc_ref[...]
    hi = lax.Precision.HIGHEST
    v = jnp.dot(svec, Wa, precision=hi, preferred_element_type=jnp.float32)
    w1 = jnp.dot(v[:, :D], W, precision=hi, preferred_element_type=jnp.float32)
    w2 = jnp.dot(v[:, D:], W, precision=hi, preferred_element_type=jnp.float32)
    wt = jnp.dot(stgt_ref[...], W, precision=hi,
                 preferred_element_type=jnp.float32)
    c = jnp.sum(svec * ba_ref[...])
    out_ref[:, 0:D] = w1
    out_ref[:, D:2 * D] = w2
    out_ref[:, 2 * D:3 * D] = wt
    out_ref[:, 3 * D:] = jnp.zeros((1, D), jnp.float32) + c


def _tc_body(nodes_ref, neigh_ref, asp_ref, W_ref, Wa_ref, ba_ref, ssrc_ref,
             stgt_ref, bias_ref, out_ref):
    W = W_ref[...]
    Wa = Wa_ref[...]
    svec = ssrc_ref[...]
    hi = lax.Precision.HIGHEST
    v = jnp.dot(svec, Wa, precision=hi, preferred_element_type=jnp.float32)
    w1 = jnp.dot(v[:, :D], W, precision=hi, preferred_element_type=jnp.float32)
    w2 = jnp.dot(v[:, D:], W, precision=hi, preferred_element_type=jnp.float32)
    wt = jnp.dot(stgt_ref[...], W, precision=hi,
                 preferred_element_type=jnp.float32)
    c = jnp.sum(svec * ba_ref[...])

    neigh = neigh_ref[...]
    asp = asp_ref[...]
    nodes = nodes_ref[...]

    s1 = jnp.sum(neigh * w1[0][None, None, :], axis=-1)
    s2 = jnp.sum(asp * w2[0][None, None, :], axis=-1)
    st = jnp.sum(nodes * wt, axis=-1)

    scores = s1 + s2 + c + st[:, None]
    scores = jnp.where(scores >= 0, scores, 0.2 * scores)
    e = jnp.exp(scores)
    attn = e / (jnp.sum(e, axis=1, keepdims=True) + 1e-16)

    agg = jnp.sum(neigh * attn[..., None], axis=1)
    out = lax.dot_general(agg, W, (((1,), (1,)), ((), ())),
                          precision=hi, preferred_element_type=jnp.float32)
    out = out + bias_ref[...]
    out_ref[...] = jnp.where(out > 0, out, jnp.exp(jnp.minimum(out, 0.0)) - 1.0)


def _hsum(x):
    idx = lax.broadcasted_iota(jnp.int32, (16,), 0)
    for m in (1, 2, 4, 8):
        x = x + x.at[idx ^ m].get(mode="promise_in_bounds")
    return x


def _sc_body(neigh_hbm, asp_hbm, nodes_hbm, params_hbm, out_hbm,
             pv, nb, ab, xb, tb, rb, ob):
    wid = lax.axis_index("s") * 2 + lax.axis_index("c")
    base = wid * NPW
    pltpu.sync_copy(params_hbm, pv)
    lanes = lax.broadcasted_iota(jnp.int32, (16,), 0)

    def chunk_body(ci, carry):
        nbase = base + ci * CH
        pltpu.sync_copy(neigh_hbm.at[pl.ds(NT + nbase, CH)], nb)
        pltpu.sync_copy(asp_hbm.at[pl.ds(NT + nbase, CH)], ab)
        pltpu.sync_copy(nodes_hbm.at[pl.ds(NT + nbase, CH)], xb)

        @plsc.parallel_loop(0, CH, unroll=2)
        def node_body(i):
            tbo = i * 256
            rbo = i * 512

            q = xb[i, pl.ds(0, 16)] * pv[pl.ds(2 * D, 16)]
            for cc in range(1, 8):
                q = q + xb[i, pl.ds(16 * cc, 16)] * pv[pl.ds(2 * D + 16 * cc, 16)]
            st = _hsum(q)

            evecs = []
            for g in range(2):
                for e in range(16):
                    k = g * 16 + e
                    p = (nb[i, k, pl.ds(0, 16)] * pv[pl.ds(0, 16)]
                         + ab[i, k, pl.ds(0, 16)] * pv[pl.ds(D, 16)])
                    for cc in range(1, 8):
                        p = (p + nb[i, k, pl.ds(16 * cc, 16)] * pv[pl.ds(16 * cc, 16)]
                             + ab[i, k, pl.ds(16 * cc, 16)] * pv[pl.ds(D + 16 * cc, 16)])
                    plsc.store_scatter(tb, [tbo + lanes * 16 + e], p)
                sg = tb[pl.ds(tbo, 16)]
                for j in range(1, 16):
                    sg = sg + tb[pl.ds(tbo + 16 * j, 16)]
                sg = sg + st + pv[pl.ds(3 * D, 16)]
                sg = jnp.where(sg >= 0.0, sg, 0.2 * sg)
                evecs.append(jnp.exp(sg))
            denom = _hsum(evecs[0] + evecs[1]) + 1e-16
            attn = [evecs[0] / denom, evecs[1] / denom]

            for g in range(2):
                for j in range(16):
                    plsc.store_scatter(
                        rb, [rbo + lanes * 16 + (g * 256 + j)], attn[g])
            acc = [None] * 8
            for k in range(K):
                a = rb[pl.ds(rbo + 16 * k, 16)]
                for cc in range(8):
                    term = a * nb[i, k, pl.ds(16 * cc, 16)]
                    acc[cc] = term if k == 0 else acc[cc] + term
            for cc in range(8):
                ob[i, pl.ds(16 * cc, 16)] = acc[cc]
        pltpu.sync_copy(ob, out_hbm.at[pl.ds(nbase, CH)])
        return carry

    lax.fori_loop(0, NCH, chunk_body, 0)


def _final_body(agg_ref, W_ref, bias_ref, out_ref):
    out = lax.dot_general(agg_ref[...], W_ref[...], (((1,), (1,)), ((), ())),
                          precision=lax.Precision.HIGHEST,
                          preferred_element_type=jnp.float32)
    out = out + bias_ref[...]
    out_ref[...] = jnp.where(out > 0, out, jnp.exp(jnp.minimum(out, 0.0)) - 1.0)


@jax.jit
def kernel(nodes, neighbors, aspects, W, Wa, ba, s_src, s_tgt, bias):
    ba2 = ba.reshape(1, D)
    ssrc2 = s_src.reshape(1, D)
    stgt2 = s_tgt.reshape(1, D)
    bias2 = bias.reshape(1, D)

    params = pl.pallas_call(
        _prep_body,
        in_specs=[pl.BlockSpec((D, D), lambda: (0, 0)),
                  pl.BlockSpec((D, 2 * D), lambda: (0, 0)),
                  pl.BlockSpec((1, D), lambda: (0, 0)),
                  pl.BlockSpec((1, D), lambda: (0, 0)),
                  pl.BlockSpec((1, D), lambda: (0, 0))],
        out_specs=pl.BlockSpec((1, PL_), lambda: (0, 0)),
        out_shape=jax.ShapeDtypeStruct((1, PL_), jnp.float32),
    )(W, Wa, ssrc2, stgt2, ba2)

    mesh = plsc.VectorSubcoreMesh(core_axis_name="c", subcore_axis_name="s")
    agg_sc = functools.partial(
        pl.kernel,
        mesh=mesh,
        compiler_params=pltpu.CompilerParams(needs_layout_passes=False),
        cost_estimate=pl.CostEstimate(
            flops=50_000_000, bytes_accessed=40_000_000, transcendentals=10_000),
        out_type=jax.ShapeDtypeStruct((NS, D), jnp.float32),
        scratch_types=[
            pltpu.VMEM((PL_,), jnp.float32),
            pltpu.VMEM((CH, K, D), jnp.float32),
            pltpu.VMEM((CH, K, D), jnp.float32),
            pltpu.VMEM((CH, D), jnp.float32),
            pltpu.VMEM((CH * 256,), jnp.float32),
            pltpu.VMEM((CH * 512,), jnp.float32),
            pltpu.VMEM((CH, D), jnp.float32),
        ],
    )(_sc_body)(neighbors, aspects, nodes, params.reshape(PL_))

    out_tc = pl.pallas_call(
        _tc_body,
        grid=(NT // BLK,),
        in_specs=[
            pl.BlockSpec((BLK, D), lambda i: (i, 0)),
            pl.BlockSpec((BLK, K, D), lambda i: (i, 0, 0)),
            pl.BlockSpec((BLK, K, D), lambda i: (i, 0, 0)),
            pl.BlockSpec((D, D), lambda i: (0, 0)),
            pl.BlockSpec((D, 2 * D), lambda i: (0, 0)),
            pl.BlockSpec((1, D), lambda i: (0, 0)),
            pl.BlockSpec((1, D), lambda i: (0, 0)),
            pl.BlockSpec((1, D), lambda i: (0, 0)),
            pl.BlockSpec((1, D), lambda i: (0, 0)),
        ],
        out_specs=pl.BlockSpec((BLK, D), lambda i: (i, 0)),
        out_shape=jax.ShapeDtypeStruct((NT, D), jnp.float32),
    )(nodes, neighbors, aspects, W, Wa, ba2, ssrc2, stgt2, bias2)

    out_sc = pl.pallas_call(
        _final_body,
        in_specs=[pl.BlockSpec((NS, D), lambda: (0, 0)),
                  pl.BlockSpec((D, D), lambda: (0, 0)),
                  pl.BlockSpec((1, D), lambda: (0, 0))],
        out_specs=pl.BlockSpec((NS, D), lambda: (0, 0)),
        out_shape=jax.ShapeDtypeStruct((NS, D), jnp.float32),
    )(agg_sc, W, bias2)

    return jnp.concatenate([out_tc, out_sc], axis=0)


# speedup vs baseline: 1.4499x; 1.4499x over previous
import functools

import jax
import jax.numpy as jnp
from jax import lax
from jax.experimental import pallas as pl
from jax.experimental.pallas import tpu as pltpu
from jax.experimental.pallas import tpu_sc as plsc

N, K, D = 1024, 32, 128
BLK = 128
NT = 896
NS = N - NT
NW = 32
NPW = NS // NW
CH = 4
NCH = NPW // CH
PL_ = 512


def _prep_body(W_ref, Wa_ref, ssrc_ref, stgt_ref, ba_ref, out_ref):
    W = W_ref[...]
    Wa = Wa_ref[...]
    svec = ssrc_ref[...]
    hi = lax.Precision.HIGHEST
    v = jnp.dot(svec, Wa, precision=hi, preferred_element_type=jnp.float32)
    w1 = jnp.dot(v[:, :D], W, precision=hi, preferred_element_type=jnp.float32)
    w2 = jnp.dot(v[:, D:], W, precision=hi, preferred_element_type=jnp.float32)
    wt = jnp.dot(stgt_ref[...], W, precision=hi,
                 preferred_element_type=jnp.float32)
    c = jnp.sum(svec * ba_ref[...])
    out_ref[:, 0:D] = w1
    out_ref[:, D:2 * D] = w2
    out_ref[:, 2 * D:3 * D] = wt
    out_ref[:, 3 * D:] = jnp.zeros((1, D), jnp.float32) + c


def _tc_body(nodes_ref, neigh_ref, asp_ref, W_ref, params_ref, bias_ref,
             out_ref):
    W = W_ref[...]
    hi = lax.Precision.HIGHEST
    w1 = params_ref[:, 0:D]
    w2 = params_ref[:, D:2 * D]
    wt = params_ref[:, 2 * D:3 * D]
    c = params_ref[:, 3 * D:3 * D + 1]

    neigh = neigh_ref[...]
    asp = asp_ref[...]
    nodes = nodes_ref[...]

    s1 = jnp.sum(neigh * w1[0][None, None, :], axis=-1)
    s2 = jnp.sum(asp * w2[0][None, None, :], axis=-1)
    st = jnp.sum(nodes * wt, axis=-1)

    scores = s1 + s2 + c + st[:, None]
    scores = jnp.where(scores >= 0, scores, 0.2 * scores)
    e = jnp.exp(scores)
    attn = e / (jnp.sum(e, axis=1, keepdims=True) + 1e-16)

    agg = jnp.sum(neigh * attn[..., None], axis=1)
    out = lax.dot_general(agg, W, (((1,), (1,)), ((), ())),
                          precision=hi, preferred_element_type=jnp.float32)
    out = out + bias_ref[...]
    out_ref[...] = jnp.where(out > 0, out, jnp.exp(jnp.minimum(out, 0.0)) - 1.0)


def _hsum(x):
    idx = lax.broadcasted_iota(jnp.int32, (16,), 0)
    for m in (1, 2, 4, 8):
        x = x + x.at[idx ^ m].get(mode="promise_in_bounds")
    return x


def _sc_body(neigh_hbm, asp_hbm, nodes_hbm, params_hbm, out_hbm,
             pv, nb, ab, xb, tb, rb, ob):
    wid = lax.axis_index("s") * 2 + lax.axis_index("c")
    base = wid * NPW
    pltpu.sync_copy(params_hbm, pv)
    lanes = lax.broadcasted_iota(jnp.int32, (16,), 0)

    def chunk_body(ci, carry):
        nbase = base + ci * CH
        pltpu.sync_copy(neigh_hbm.at[pl.ds(NT + nbase, CH)], nb)
        pltpu.sync_copy(asp_hbm.at[pl.ds(NT + nbase, CH)], ab)
        pltpu.sync_copy(nodes_hbm.at[pl.ds(NT + nbase, CH)], xb)

        def node_body(i, carry2):
            tbo = i * 256
            rbo = i * 512

            q = xb[i, pl.ds(0, 16)] * pv[pl.ds(2 * D, 16)]
            for cc in range(1, 8):
                q = q + xb[i, pl.ds(16 * cc, 16)] * pv[pl.ds(2 * D + 16 * cc, 16)]
            st = _hsum(q)

            evecs = []
            for g in range(2):
                for e in range(16):
                    k = g * 16 + e
                    p = (nb[i, k, pl.ds(0, 16)] * pv[pl.ds(0, 16)]
                         + ab[i, k, pl.ds(0, 16)] * pv[pl.ds(D, 16)])
                    for cc in range(1, 8):
                        p = (p + nb[i, k, pl.ds(16 * cc, 16)] * pv[pl.ds(16 * cc, 16)]
                             + ab[i, k, pl.ds(16 * cc, 16)] * pv[pl.ds(D + 16 * cc, 16)])
                    plsc.store_scatter(tb, [tbo + lanes * 16 + e], p)
                sg = tb[pl.ds(tbo, 16)]
                for j in range(1, 16):
                    sg = sg + tb[pl.ds(tbo + 16 * j, 16)]
                sg = sg + st + pv[pl.ds(3 * D, 16)]
                sg = jnp.where(sg >= 0.0, sg, 0.2 * sg)
                evecs.append(jnp.exp(sg))
            denom = _hsum(evecs[0] + evecs[1]) + 1e-16
            attn = [evecs[0] / denom, evecs[1] / denom]

            for g in range(2):
                for j in range(16):
                    plsc.store_scatter(
                        rb, [rbo + lanes * 16 + (g * 256 + j)], attn[g])
            acc = [None] * 8
            for k in range(K):
                a = rb[pl.ds(rbo + 16 * k, 16)]
                for cc in range(8):
                    term = a * nb[i, k, pl.ds(16 * cc, 16)]
                    acc[cc] = term if k == 0 else acc[cc] + term
            for cc in range(8):
                ob[i, pl.ds(16 * cc, 16)] = acc[cc]
            return carry2

        lax.fori_loop(0, CH, node_body, 0)
        pltpu.sync_copy(ob, out_hbm.at[pl.ds(nbase, CH)])
        return carry

    lax.fori_loop(0, NCH, chunk_body, 0)


def _final_body(agg_ref, W_ref, bias_ref, out_ref):
    out = lax.dot_general(agg_ref[...], W_ref[...], (((1,), (1,)), ((), ())),
                          precision=lax.Precision.HIGHEST,
                          preferred_element_type=jnp.float32)
    out = out + bias_ref[...]
    out_ref[...] = jnp.where(out > 0, out, jnp.exp(jnp.minimum(out, 0.0)) - 1.0)


@jax.jit
def kernel(nodes, neighbors, aspects, W, Wa, ba, s_src, s_tgt, bias):
    ba2 = ba.reshape(1, D)
    ssrc2 = s_src.reshape(1, D)
    stgt2 = s_tgt.reshape(1, D)
    bias2 = bias.reshape(1, D)

    params = pl.pallas_call(
        _prep_body,
        in_specs=[pl.BlockSpec((D, D), lambda: (0, 0)),
                  pl.BlockSpec((D, 2 * D), lambda: (0, 0)),
                  pl.BlockSpec((1, D), lambda: (0, 0)),
                  pl.BlockSpec((1, D), lambda: (0, 0)),
                  pl.BlockSpec((1, D), lambda: (0, 0))],
        out_specs=pl.BlockSpec((1, PL_), lambda: (0, 0)),
        out_shape=jax.ShapeDtypeStruct((1, PL_), jnp.float32),
    )(W, Wa, ssrc2, stgt2, ba2)

    mesh = plsc.VectorSubcoreMesh(core_axis_name="c", subcore_axis_name="s")
    agg_sc = functools.partial(
        pl.kernel,
        mesh=mesh,
        compiler_params=pltpu.CompilerParams(needs_layout_passes=False),
        cost_estimate=pl.CostEstimate(
            flops=50_000_000, bytes_accessed=40_000_000, transcendentals=10_000),
        out_type=jax.ShapeDtypeStruct((NS, D), jnp.float32),
        scratch_types=[
            pltpu.VMEM((PL_,), jnp.float32),
            pltpu.VMEM((CH, K, D), jnp.float32),
            pltpu.VMEM((CH, K, D), jnp.float32),
            pltpu.VMEM((CH, D), jnp.float32),
            pltpu.VMEM((CH * 256,), jnp.float32),
            pltpu.VMEM((CH * 512,), jnp.float32),
            pltpu.VMEM((CH, D), jnp.float32),
        ],
    )(_sc_body)(neighbors, aspects, nodes, params.reshape(PL_))

    out_tc = pl.pallas_call(
        _tc_body,
        grid=(NT // BLK,),
        in_specs=[
            pl.BlockSpec((BLK, D), lambda i: (i, 0)),
            pl.BlockSpec((BLK, K, D), lambda i: (i, 0, 0)),
            pl.BlockSpec((BLK, K, D), lambda i: (i, 0, 0)),
            pl.BlockSpec((D, D), lambda i: (0, 0)),
            pl.BlockSpec((1, PL_), lambda i: (0, 0)),
            pl.BlockSpec((1, D), lambda i: (0, 0)),
        ],
        out_specs=pl.BlockSpec((BLK, D), lambda i: (i, 0)),
        out_shape=jax.ShapeDtypeStruct((NT, D), jnp.float32),
    )(nodes, neighbors, aspects, W, params, bias2)

    out_sc = pl.pallas_call(
        _final_body,
        in_specs=[pl.BlockSpec((NS, D), lambda: (0, 0)),
                  pl.BlockSpec((D, D), lambda: (0, 0)),
                  pl.BlockSpec((1, D), lambda: (0, 0))],
        out_specs=pl.BlockSpec((NS, D), lambda: (0, 0)),
        out_shape=jax.ShapeDtypeStruct((NS, D), jnp.float32),
    )(agg_sc, W, bias2)

    return jnp.concatenate([out_tc, out_sc], axis=0)
